# Initial kernel scaffold; baseline (speedup 1.0000x reference)
import jax
import jax.numpy as jnp
from jax.experimental import pallas as pl


def kernel(Z, hyperedge_index, num_nodes):
    raise NotImplementedError("write your pallas kernel here")


# speedup vs baseline: 47.0257x; 47.0257x over previous
import jax
import jax.numpy as jnp
from jax import lax
from jax.experimental import pallas as pl
from jax.experimental.pallas import tpu as pltpu
from jax.experimental.pallas import tpu_sc as plsc

NC, NS = 2, 16
NV = 100000
NP = 100352
E = 3200000
K = 16
W = 128
VB = 56
EPR = 25088
EP = EPR * W
_ZCH = NP // NS

_sc_mesh = dict(core_axis_name="c", subcore_axis_name="s",
                num_cores=NC, num_subcores=NS)
_sc_params = pltpu.CompilerParams(use_tc_tiling_on_sc=False)

_H_TROWS = EPR // NS
_H_BATCH = _H_TROWS // VB


def _hist_body(nidx_hbm, eidx_hbm, out_hbm, idx_v, ones_v, zb_v, bins):
    c = lax.axis_index("c")
    s = lax.axis_index("s")

    def _fill_ones(i, _):
        ones_v[pl.ds(i * 16, 16)] = jnp.ones((16,), jnp.float32)
        return 0

    lax.fori_loop(0, W // 16, _fill_ones, 0)

    def _fill_zeros(i, _):
        zb_v[pl.ds(i * 16, 16)] = jnp.zeros((16,), jnp.float32)
        return 0

    lax.fori_loop(0, _ZCH // 16, _fill_zeros, 0)

    zbase = pl.multiple_of(s * _ZCH, 128)
    pltpu.sync_copy(zb_v, bins.at[pl.ds(zbase, _ZCH)])
    plsc.subcore_barrier()

    def _accumulate(src_hbm):
        for b in range(_H_BATCH):
            r0 = pl.multiple_of(s * _H_TROWS + b * VB, 8)
            pltpu.sync_copy(src_hbm.at[pl.ds(r0, VB), :], idx_v)

            def _one_row(t, _):
                pltpu.sync_copy(ones_v, bins.at[idx_v.at[t]], add=True)
                return 0

            lax.fori_loop(0, VB, _one_row, 0)

    @pl.when(c == 0)
    def _():
        _accumulate(nidx_hbm)

    @pl.when(c == 1)
    def _():
        _accumulate(eidx_hbm)

    plsc.subcore_barrier()
    pltpu.sync_copy(bins.at[pl.ds(zbase, _ZCH)], out_hbm.at[c, 0, pl.ds(zbase, _ZCH)])


@jax.jit
def _hist_call(nidx2, eidx2):
    return pl.kernel(
        _hist_body,
        out_type=jax.ShapeDtypeStruct((NC, 1, NP), jnp.float32),
        mesh=plsc.VectorSubcoreMesh(**_sc_mesh),
        compiler_params=_sc_params,
        scratch_types=[
            pltpu.VMEM((VB, W), jnp.int32),
            pltpu.VMEM((W,), jnp.float32),
            pltpu.VMEM((_ZCH,), jnp.float32),
            pltpu.VMEM_SHARED((NP,), jnp.float32),
        ],
    )(nidx2, eidx2)


_P_ROWS = 3136
_P_GRID = NP // _P_ROWS


def _prep_body(z_ref, dv_ref, y_ref, den_ref, sza_ref):
    i = pl.program_id(0)
    rid = i * _P_ROWS + lax.broadcasted_iota(jnp.int32, (_P_ROWS, 1), 0)
    valid = rid < NV
    z = jnp.where(valid, z_ref[...], 0.0)
    dv = dv_ref[...]
    dvc = jnp.maximum(dv, 1.0)
    y_ref[...] = z * lax.rsqrt(dvc)
    z2 = z * z
    den_p = jnp.sum(z2 * dvc, axis=0, keepdims=True)
    sza_p = jnp.sum(jnp.where(dv > 0.0, z2, 0.0), axis=0, keepdims=True)

    @pl.when(i == 0)
    def _():
        den_ref[...] = den_p
        sza_ref[...] = sza_p

    @pl.when(i > 0)
    def _():
        den_ref[...] += den_p
        sza_ref[...] += sza_p


@jax.jit
def _prep_call(z, dv2d):
    return pl.pallas_call(
        _prep_body,
        grid=(_P_GRID,),
        in_specs=[
            pl.BlockSpec((_P_ROWS, K), lambda i: (i, 0)),
            pl.BlockSpec((_P_ROWS, 1), lambda i: (i, 0)),
        ],
        out_specs=[
            pl.BlockSpec((_P_ROWS, K), lambda i: (i, 0)),
            pl.BlockSpec((1, K), lambda i: (0, 0)),
            pl.BlockSpec((1, K), lambda i: (0, 0)),
        ],
        out_shape=[
            jax.ShapeDtypeStruct((NP, K), jnp.float32),
            jax.ShapeDtypeStruct((1, K), jnp.float32),
            jax.ShapeDtypeStruct((1, K), jnp.float32),
        ],
    )(z, dv2d)


_S_TROWS = EPR // (NC * NS)
_S_BATCH = _S_TROWS // VB
_S_ZROWS = 784


def _scatter_body(y_hbm, nidx_hbm, eidx_hbm, out_hbm, nv, ev, rows, zb, sem, acc):
    c = lax.axis_index("c")
    s = lax.axis_index("s")
    g = c * NS + s

    def _fill_zeros(i, _):
        zb[i, :] = jnp.zeros((16,), jnp.float32)
        return 0

    lax.fori_loop(0, _S_ZROWS, _fill_zeros, 0)
    for t in range(8):
        r0 = pl.multiple_of(s * _ZCH + t * _S_ZROWS, 8)
        pltpu.sync_copy(zb, acc.at[pl.ds(r0, _S_ZROWS), :])
    plsc.subcore_barrier()

    base = g * _S_TROWS
    for b in range(_S_BATCH):
        r0 = pl.multiple_of(base + b * VB, 8)
        pltpu.sync_copy(nidx_hbm.at[pl.ds(r0, VB), :], nv)
        pltpu.sync_copy(eidx_hbm.at[pl.ds(r0, VB), :], ev)

        def _one_row(t, _):
            pltpu.async_copy(y_hbm.at[nv.at[t]], rows, sem).wait()
            pltpu.sync_copy(rows, acc.at[ev.at[t]], add=True)
            return 0

        lax.fori_loop(0, VB, _one_row, 0)

    plsc.subcore_barrier()
    for t in range(8):
        r0 = pl.multiple_of(s * _ZCH + t * _S_ZROWS, 8)
        pltpu.sync_copy(acc.at[pl.ds(r0, _S_ZROWS), :],
                        out_hbm.at[c, pl.ds(r0, _S_ZROWS), :])


@jax.jit
def _scatter_call(y, nidx2, eidx2):
    return pl.kernel(
        _scatter_body,
        out_type=jax.ShapeDtypeStruct((NC, NP, K), jnp.float32),
        mesh=plsc.VectorSubcoreMesh(**_sc_mesh),
        compiler_params=_sc_params,
        scratch_types=[
            pltpu.VMEM((VB, W), jnp.int32),
            pltpu.VMEM((VB, W), jnp.int32),
            pltpu.VMEM((W, K), jnp.float32),
            pltpu.VMEM((_S_ZROWS, K), jnp.float32),
            pltpu.SemaphoreType.DMA,
            pltpu.VMEM_SHARED((NP, K), jnp.float32),
        ],
    )(y, nidx2, eidx2)


_F_ROWS = 3136
_F_GRID = NP // _F_ROWS


def _final_body(p0_ref, p1_ref, de_ref, den_ref, sza_ref, out_ref, acc_ref):
    i = pl.program_id(0)

    @pl.when(i == 0)
    def _():
        acc_ref[...] = jnp.zeros_like(acc_ref)

    ys = p0_ref[...] + p1_ref[...]
    de = jnp.maximum(de_ref[...], 1.0)
    acc_ref[...] += jnp.sum(ys * ys / de, axis=0, keepdims=True)

    @pl.when(i == pl.num_programs(0) - 1)
    def _():
        num = sza_ref[...] - acc_ref[...]
        rq = num / (den_ref[...] + 1e-8)
        out_ref[...] = jnp.mean(rq).reshape(1, 1)


@jax.jit
def _final_call(p0, p1, de2d, den, sza):
    return pl.pallas_call(
        _final_body,
        grid=(_F_GRID,),
        in_specs=[
            pl.BlockSpec((_F_ROWS, K), lambda i: (i, 0)),
            pl.BlockSpec((_F_ROWS, K), lambda i: (i, 0)),
            pl.BlockSpec((_F_ROWS, 1), lambda i: (i, 0)),
            pl.BlockSpec((1, K), lambda i: (0, 0)),
            pl.BlockSpec((1, K), lambda i: (0, 0)),
        ],
        out_specs=pl.BlockSpec((1, 1), lambda i: (0, 0)),
        out_shape=jax.ShapeDtypeStruct((1, 1), jnp.float32),
        scratch_shapes=[pltpu.VMEM((1, K), jnp.float32)],
    )(p0, p1, de2d, den, sza)


def kernel(Z, hyperedge_index, num_nodes):
    del num_nodes
    pad = jnp.full((EP - E,), NV, jnp.int32)
    nidx2 = jnp.concatenate([hyperedge_index[0], pad]).reshape(EPR, W)
    eidx2 = jnp.concatenate([hyperedge_index[1], pad]).reshape(EPR, W)
    hist = _hist_call(nidx2, eidx2)
    dv2d = hist[0, 0].reshape(NP, 1)
    de2d = hist[1, 0].reshape(NP, 1)
    y, den, sza = _prep_call(Z, dv2d)
    parts = _scatter_call(y, nidx2, eidx2)
    loss = _final_call(parts[0], parts[1], de2d, den, sza)
    return loss.reshape(())


# speedup vs baseline: 73.1065x; 1.5546x over previous
import jax
import jax.numpy as jnp
from jax import lax
from jax.experimental import pallas as pl
from jax.experimental.pallas import tpu as pltpu
from jax.experimental.pallas import tpu_sc as plsc

NC, NS = 2, 16
NV = 100000
NP = 100352
E = 3200000
K = 16
W = 128
VB = 56
EPR = 25088
EP = EPR * W
_ZCH = NP // NS

_sc_mesh = dict(core_axis_name="c", subcore_axis_name="s",
                num_cores=NC, num_subcores=NS)
_sc_params = pltpu.CompilerParams(use_tc_tiling_on_sc=False)

_H_TROWS = EPR // NS
_H_BATCH = _H_TROWS // VB


def _hist_body(nidx_hbm, eidx_hbm, out_hbm, idx_v, ones_v, zb_v, bins):
    c = lax.axis_index("c")
    s = lax.axis_index("s")

    def _fill_ones(i, _):
        ones_v[pl.ds(i * 16, 16)] = jnp.ones((16,), jnp.float32)
        return 0

    lax.fori_loop(0, W // 16, _fill_ones, 0)

    def _fill_zeros(i, _):
        zb_v[pl.ds(i * 16, 16)] = jnp.zeros((16,), jnp.float32)
        return 0

    lax.fori_loop(0, _ZCH // 16, _fill_zeros, 0)

    zbase = pl.multiple_of(s * _ZCH, 128)
    pltpu.sync_copy(zb_v, bins.at[pl.ds(zbase, _ZCH)])
    plsc.subcore_barrier()

    def _accumulate(src_hbm):
        for b in range(_H_BATCH):
            r0 = pl.multiple_of(s * _H_TROWS + b * VB, 8)
            pltpu.sync_copy(src_hbm.at[pl.ds(r0, VB), :], idx_v)

            def _one_row(t, _):
                pltpu.sync_copy(ones_v, bins.at[idx_v.at[t]], add=True)
                return 0

            lax.fori_loop(0, VB, _one_row, 0)

    @pl.when(c == 0)
    def _():
        _accumulate(nidx_hbm)

    @pl.when(c == 1)
    def _():
        _accumulate(eidx_hbm)

    plsc.subcore_barrier()
    pltpu.sync_copy(bins.at[pl.ds(zbase, _ZCH)], out_hbm.at[c, 0, pl.ds(zbase, _ZCH)])


@jax.jit
def _hist_call(nidx2, eidx2):
    return pl.kernel(
        _hist_body,
        out_type=jax.ShapeDtypeStruct((NC, 1, NP), jnp.float32),
        mesh=plsc.VectorSubcoreMesh(**_sc_mesh),
        compiler_params=_sc_params,
        scratch_types=[
            pltpu.VMEM((VB, W), jnp.int32),
            pltpu.VMEM((W,), jnp.float32),
            pltpu.VMEM((_ZCH,), jnp.float32),
            pltpu.VMEM_SHARED((NP,), jnp.float32),
        ],
    )(nidx2, eidx2)


_P_ROWS = 3136
_P_GRID = NP // _P_ROWS


def _prep_body(z_ref, dv_ref, y_ref, den_ref, sza_ref):
    i = pl.program_id(0)
    rid = i * _P_ROWS + lax.broadcasted_iota(jnp.int32, (_P_ROWS, 1), 0)
    valid = rid < NV
    z = jnp.where(valid, z_ref[...], 0.0)
    dv = dv_ref[...]
    dvc = jnp.maximum(dv, 1.0)
    y_ref[...] = z * lax.rsqrt(dvc)
    z2 = z * z
    den_p = jnp.sum(z2 * dvc, axis=0, keepdims=True)
    sza_p = jnp.sum(jnp.where(dv > 0.0, z2, 0.0), axis=0, keepdims=True)

    @pl.when(i == 0)
    def _():
        den_ref[...] = den_p
        sza_ref[...] = sza_p

    @pl.when(i > 0)
    def _():
        den_ref[...] += den_p
        sza_ref[...] += sza_p


@jax.jit
def _prep_call(z, dv2d):
    return pl.pallas_call(
        _prep_body,
        grid=(_P_GRID,),
        in_specs=[
            pl.BlockSpec((_P_ROWS, K), lambda i: (i, 0)),
            pl.BlockSpec((_P_ROWS, 1), lambda i: (i, 0)),
        ],
        out_specs=[
            pl.BlockSpec((_P_ROWS, K), lambda i: (i, 0)),
            pl.BlockSpec((1, K), lambda i: (0, 0)),
            pl.BlockSpec((1, K), lambda i: (0, 0)),
        ],
        out_shape=[
            jax.ShapeDtypeStruct((NP, K), jnp.float32),
            jax.ShapeDtypeStruct((1, K), jnp.float32),
            jax.ShapeDtypeStruct((1, K), jnp.float32),
        ],
    )(z, dv2d)


_S_TROWS = EPR // (NC * NS)
_S_BATCH = _S_TROWS // VB
_S_ZROWS = 784


_NBUF = 4


def _scatter_body(y_hbm, nidx_hbm, eidx_hbm, out_hbm, nv, ev,
                  r0b, r1b, r2b, r3b, s0, s1, s2, s3, acc):
    c = lax.axis_index("c")
    s = lax.axis_index("s")
    g = c * NS + s
    rbufs = (r0b, r1b, r2b, r3b)
    sems = (s0, s1, s2, s3)

    def _fill_zeros(i, _):
        r0b[i, :] = jnp.zeros((16,), jnp.float32)
        return 0

    lax.fori_loop(0, W, _fill_zeros, 0)
    for t in range(_ZCH // W):
        z0 = pl.multiple_of(s * _ZCH + t * W, 8)
        pltpu.sync_copy(r0b, acc.at[pl.ds(z0, W), :])
    plsc.subcore_barrier()

    base = g * _S_TROWS

    def _batch(b, _):
        r0 = pl.multiple_of(base + b * VB, 8)
        pltpu.sync_copy(nidx_hbm.at[pl.ds(r0, VB), :], nv)
        pltpu.sync_copy(eidx_hbm.at[pl.ds(r0, VB), :], ev)
        for t in range(_NBUF):
            pltpu.async_copy(y_hbm.at[nv.at[t]], rbufs[t], sems[t])
        for t in range(VB):
            k = t % _NBUF
            pltpu.make_async_copy(y_hbm.at[nv.at[t]], rbufs[k], sems[k]).wait()
            pltpu.sync_copy(rbufs[k], acc.at[ev.at[t]], add=True)
            if t + _NBUF < VB:
                pltpu.async_copy(y_hbm.at[nv.at[t + _NBUF]], rbufs[k], sems[k])
        return 0

    lax.fori_loop(0, _S_BATCH, _batch, 0)

    plsc.subcore_barrier()
    for t in range(8):
        r0 = pl.multiple_of(s * _ZCH + t * _S_ZROWS, 8)
        pltpu.sync_copy(acc.at[pl.ds(r0, _S_ZROWS), :],
                        out_hbm.at[c, pl.ds(r0, _S_ZROWS), :])


@jax.jit
def _scatter_call(y, nidx2, eidx2):
    return pl.kernel(
        _scatter_body,
        out_type=jax.ShapeDtypeStruct((NC, NP, K), jnp.float32),
        mesh=plsc.VectorSubcoreMesh(**_sc_mesh),
        compiler_params=_sc_params,
        scratch_types=[
            pltpu.VMEM((VB, W), jnp.int32),
            pltpu.VMEM((VB, W), jnp.int32),
            pltpu.VMEM((W, K), jnp.float32),
            pltpu.VMEM((W, K), jnp.float32),
            pltpu.VMEM((W, K), jnp.float32),
            pltpu.VMEM((W, K), jnp.float32),
            pltpu.SemaphoreType.DMA,
            pltpu.SemaphoreType.DMA,
            pltpu.SemaphoreType.DMA,
            pltpu.SemaphoreType.DMA,
            pltpu.VMEM_SHARED((NP, K), jnp.float32),
        ],
    )(y, nidx2, eidx2)


_F_ROWS = 3136
_F_GRID = NP // _F_ROWS


def _final_body(p0_ref, p1_ref, de_ref, den_ref, sza_ref, out_ref, acc_ref):
    i = pl.program_id(0)

    @pl.when(i == 0)
    def _():
        acc_ref[...] = jnp.zeros_like(acc_ref)

    ys = p0_ref[...] + p1_ref[...]
    de = jnp.maximum(de_ref[...], 1.0)
    acc_ref[...] += jnp.sum(ys * ys / de, axis=0, keepdims=True)

    @pl.when(i == pl.num_programs(0) - 1)
    def _():
        num = sza_ref[...] - acc_ref[...]
        rq = num / (den_ref[...] + 1e-8)
        out_ref[...] = jnp.mean(rq).reshape(1, 1)


@jax.jit
def _final_call(p0, p1, de2d, den, sza):
    return pl.pallas_call(
        _final_body,
        grid=(_F_GRID,),
        in_specs=[
            pl.BlockSpec((_F_ROWS, K), lambda i: (i, 0)),
            pl.BlockSpec((_F_ROWS, K), lambda i: (i, 0)),
            pl.BlockSpec((_F_ROWS, 1), lambda i: (i, 0)),
            pl.BlockSpec((1, K), lambda i: (0, 0)),
            pl.BlockSpec((1, K), lambda i: (0, 0)),
        ],
        out_specs=pl.BlockSpec((1, 1), lambda i: (0, 0)),
        out_shape=jax.ShapeDtypeStruct((1, 1), jnp.float32),
        scratch_shapes=[pltpu.VMEM((1, K), jnp.float32)],
    )(p0, p1, de2d, den, sza)


def kernel(Z, hyperedge_index, num_nodes):
    del num_nodes
    pad = jnp.full((EP - E,), NV, jnp.int32)
    nidx2 = jnp.concatenate([hyperedge_index[0], pad]).reshape(EPR, W)
    eidx2 = jnp.concatenate([hyperedge_index[1], pad]).reshape(EPR, W)
    hist = _hist_call(nidx2, eidx2)
    dv2d = hist[0, 0].reshape(NP, 1)
    de2d = hist[1, 0].reshape(NP, 1)
    y, den, sza = _prep_call(Z, dv2d)
    parts = _scatter_call(y, nidx2, eidx2)
    loss = _final_call(parts[0], parts[1], de2d, den, sza)
    return loss.reshape(())


# speedup vs baseline: 85.5186x; 1.1698x over previous
import jax
import jax.numpy as jnp
from jax import lax
from jax.experimental import pallas as pl
from jax.experimental.pallas import tpu as pltpu
from jax.experimental.pallas import tpu_sc as plsc

NC, NS = 2, 16
NV = 100000
NP = 100352
E = 3200000
K = 16
W = 128
VB = 56
EPR = 25088
EP = EPR * W
_ZCH = NP // NS

_sc_mesh = dict(core_axis_name="c", subcore_axis_name="s",
                num_cores=NC, num_subcores=NS)
_sc_params = pltpu.CompilerParams(use_tc_tiling_on_sc=False)

_H_TROWS = EPR // NS
_H_BATCH = _H_TROWS // VB


def _hist_body(nidx_hbm, eidx_hbm, out_hbm, idx_v, ones_v, zb_v, bins):
    c = lax.axis_index("c")
    s = lax.axis_index("s")

    def _fill_ones(i, _):
        ones_v[pl.ds(i * 16, 16)] = jnp.ones((16,), jnp.float32)
        return 0

    lax.fori_loop(0, W // 16, _fill_ones, 0)

    def _fill_zeros(i, _):
        zb_v[pl.ds(i * 16, 16)] = jnp.zeros((16,), jnp.float32)
        return 0

    lax.fori_loop(0, _ZCH // 16, _fill_zeros, 0)

    zbase = pl.multiple_of(s * _ZCH, 128)
    pltpu.sync_copy(zb_v, bins.at[pl.ds(zbase, _ZCH)])
    plsc.subcore_barrier()

    def _accumulate(src_hbm):
        for b in range(_H_BATCH):
            r0 = pl.multiple_of(s * _H_TROWS + b * VB, 8)
            pltpu.sync_copy(src_hbm.at[pl.ds(r0, VB), :], idx_v)

            def _one_row(t, _):
                pltpu.sync_copy(ones_v, bins.at[idx_v.at[t]], add=True)
                return 0

            lax.fori_loop(0, VB, _one_row, 0)

    @pl.when(c == 0)
    def _():
        _accumulate(nidx_hbm)

    @pl.when(c == 1)
    def _():
        _accumulate(eidx_hbm)

    plsc.subcore_barrier()
    pltpu.sync_copy(bins.at[pl.ds(zbase, _ZCH)], out_hbm.at[c, 0, pl.ds(zbase, _ZCH)])


@jax.jit
def _hist_call(nidx2, eidx2):
    return pl.kernel(
        _hist_body,
        out_type=jax.ShapeDtypeStruct((NC, 1, NP), jnp.float32),
        mesh=plsc.VectorSubcoreMesh(**_sc_mesh),
        compiler_params=_sc_params,
        scratch_types=[
            pltpu.VMEM((VB, W), jnp.int32),
            pltpu.VMEM((W,), jnp.float32),
            pltpu.VMEM((_ZCH,), jnp.float32),
            pltpu.VMEM_SHARED((NP,), jnp.float32),
        ],
    )(nidx2, eidx2)


_GROUPS = NP // 128
_P_GB = 56
_P_GRID = _GROUPS // _P_GB


def _sel(q):
    m = lax.broadcasted_iota(jnp.int32, (128, 128), 0)
    l = lax.broadcasted_iota(jnp.int32, (128, 128), 1)
    return (m == 8 * q + l // 16).astype(jnp.float32)


def _prep_body(z_ref, dv_ref, y_ref, den_ref, sza_ref):
    i = pl.program_id(0)
    dv = dv_ref[...]
    dvc = jnp.maximum(dv, 1.0)
    scale = lax.rsqrt(dvc)
    act = (dv > 0.0).astype(jnp.float32)
    den_p = jnp.zeros((1, 128), jnp.float32)
    sza_p = jnp.zeros((1, 128), jnp.float32)
    for q in range(16):
        sq = _sel(q)
        z = z_ref[:, q, :]
        z2 = z * z
        y_ref[:, q, :] = z * jnp.dot(scale, sq, preferred_element_type=jnp.float32)
        den_p += jnp.sum(z2 * jnp.dot(dvc, sq, preferred_element_type=jnp.float32),
                         axis=0, keepdims=True)
        sza_p += jnp.sum(z2 * jnp.dot(act, sq, preferred_element_type=jnp.float32),
                         axis=0, keepdims=True)

    @pl.when(i == 0)
    def _():
        den_ref[...] = den_p
        sza_ref[...] = sza_p

    @pl.when(i > 0)
    def _():
        den_ref[...] += den_p
        sza_ref[...] += sza_p


@jax.jit
def _prep_call(z4, dvp):
    return pl.pallas_call(
        _prep_body,
        grid=(_P_GRID,),
        in_specs=[
            pl.BlockSpec((_P_GB, 16, 128), lambda i: (i, 0, 0)),
            pl.BlockSpec((_P_GB, 128), lambda i: (i, 0)),
        ],
        out_specs=[
            pl.BlockSpec((_P_GB, 16, 128), lambda i: (i, 0, 0)),
            pl.BlockSpec((1, 128), lambda i: (0, 0)),
            pl.BlockSpec((1, 128), lambda i: (0, 0)),
        ],
        out_shape=[
            jax.ShapeDtypeStruct((_GROUPS, 16, 128), jnp.float32),
            jax.ShapeDtypeStruct((1, 128), jnp.float32),
            jax.ShapeDtypeStruct((1, 128), jnp.float32),
        ],
    )(z4, dvp)


_S_TROWS = EPR // (NC * NS)
_S_BATCH = _S_TROWS // VB
_S_ZROWS = 784


_NBUF = 4


def _scatter_body(y_hbm, nidx_hbm, eidx_hbm, out_hbm, nv, ev,
                  r0b, r1b, r2b, r3b, s0, s1, s2, s3, acc):
    c = lax.axis_index("c")
    s = lax.axis_index("s")
    g = c * NS + s
    rbufs = (r0b, r1b, r2b, r3b)
    sems = (s0, s1, s2, s3)

    def _fill_zeros(i, _):
        r0b[i, :] = jnp.zeros((16,), jnp.float32)
        return 0

    lax.fori_loop(0, W, _fill_zeros, 0)
    for t in range(_ZCH // W):
        z0 = pl.multiple_of(s * _ZCH + t * W, 8)
        pltpu.sync_copy(r0b, acc.at[pl.ds(z0, W), :])
    plsc.subcore_barrier()

    base = g * _S_TROWS

    def _batch(b, _):
        r0 = pl.multiple_of(base + b * VB, 8)
        pltpu.sync_copy(nidx_hbm.at[pl.ds(r0, VB), :], nv)
        pltpu.sync_copy(eidx_hbm.at[pl.ds(r0, VB), :], ev)
        for t in range(_NBUF):
            pltpu.async_copy(y_hbm.at[nv.at[t]], rbufs[t], sems[t])
        for t in range(VB):
            k = t % _NBUF
            pltpu.make_async_copy(y_hbm.at[nv.at[t]], rbufs[k], sems[k]).wait()
            pltpu.sync_copy(rbufs[k], acc.at[ev.at[t]], add=True)
            if t + _NBUF < VB:
                pltpu.async_copy(y_hbm.at[nv.at[t + _NBUF]], rbufs[k], sems[k])
        return 0

    lax.fori_loop(0, _S_BATCH, _batch, 0)

    plsc.subcore_barrier()
    for t in range(8):
        r0 = pl.multiple_of(s * _ZCH + t * _S_ZROWS, 8)
        pltpu.sync_copy(acc.at[pl.ds(r0, _S_ZROWS), :],
                        out_hbm.at[c, pl.ds(r0, _S_ZROWS), :])


@jax.jit
def _scatter_call(y, nidx2, eidx2):
    return pl.kernel(
        _scatter_body,
        out_type=jax.ShapeDtypeStruct((NC, NP, K), jnp.float32),
        mesh=plsc.VectorSubcoreMesh(**_sc_mesh),
        compiler_params=_sc_params,
        scratch_types=[
            pltpu.VMEM((VB, W), jnp.int32),
            pltpu.VMEM((VB, W), jnp.int32),
            pltpu.VMEM((W, K), jnp.float32),
            pltpu.VMEM((W, K), jnp.float32),
            pltpu.VMEM((W, K), jnp.float32),
            pltpu.VMEM((W, K), jnp.float32),
            pltpu.SemaphoreType.DMA,
            pltpu.SemaphoreType.DMA,
            pltpu.SemaphoreType.DMA,
            pltpu.SemaphoreType.DMA,
            pltpu.VMEM_SHARED((NP, K), jnp.float32),
        ],
    )(y, nidx2, eidx2)


_F_GB = 56
_F_GRID = _GROUPS // _F_GB


def _final_body(p0_ref, p1_ref, de_ref, den_ref, sza_ref, out_ref, acc_ref):
    i = pl.program_id(0)

    @pl.when(i == 0)
    def _():
        acc_ref[...] = jnp.zeros_like(acc_ref)

    de = jnp.maximum(de_ref[...], 1.0)
    part = jnp.zeros((1, 128), jnp.float32)
    for q in range(16):
        de16 = jnp.dot(de, _sel(q), preferred_element_type=jnp.float32)
        ys = p0_ref[:, q, :] + p1_ref[:, q, :]
        part += jnp.sum(ys * ys / de16, axis=0, keepdims=True)
    acc_ref[...] += part

    @pl.when(i == pl.num_programs(0) - 1)
    def _():
        lf = lax.broadcasted_iota(jnp.int32, (128, 16), 0)
        cf = lax.broadcasted_iota(jnp.int32, (128, 16), 1)
        fold = (lf % 16 == cf).astype(jnp.float32)
        num16 = jnp.dot(sza_ref[...] - acc_ref[...], fold,
                        preferred_element_type=jnp.float32)
        den16 = jnp.dot(den_ref[...], fold, preferred_element_type=jnp.float32)
        rq = num16 / (den16 + 1e-8)
        out_ref[...] = (jnp.sum(rq) / 16.0).reshape(1, 1)


@jax.jit
def _final_call(p0, p1, dep, den8, sza8):
    return pl.pallas_call(
        _final_body,
        grid=(_F_GRID,),
        in_specs=[
            pl.BlockSpec((_F_GB, 16, 128), lambda i: (i, 0, 0)),
            pl.BlockSpec((_F_GB, 16, 128), lambda i: (i, 0, 0)),
            pl.BlockSpec((_F_GB, 128), lambda i: (i, 0)),
            pl.BlockSpec((1, 128), lambda i: (0, 0)),
            pl.BlockSpec((1, 128), lambda i: (0, 0)),
        ],
        out_specs=pl.BlockSpec((1, 1), lambda i: (0, 0)),
        out_shape=jax.ShapeDtypeStruct((1, 1), jnp.float32),
        scratch_shapes=[pltpu.VMEM((1, 128), jnp.float32)],
    )(p0, p1, dep, den8, sza8)


def kernel(Z, hyperedge_index, num_nodes):
    del num_nodes
    pad = jnp.full((EP - E,), NV, jnp.int32)
    nidx2 = jnp.concatenate([hyperedge_index[0], pad]).reshape(EPR, W)
    eidx2 = jnp.concatenate([hyperedge_index[1], pad]).reshape(EPR, W)
    hist = _hist_call(nidx2, eidx2)
    dvp = hist[0, 0].reshape(_GROUPS, 128)
    dep = hist[1, 0].reshape(_GROUPS, 128)
    z4 = jnp.pad(Z, ((0, NP - NV), (0, 0))).reshape(_GROUPS, 16, 128)
    y4, den8, sza8 = _prep_call(z4, dvp)
    y = y4.reshape(NP, K)
    parts = _scatter_call(y, nidx2, eidx2)
    parts4 = parts.reshape(NC, _GROUPS, 16, 128)
    loss = _final_call(parts4[0], parts4[1], dep, den8, sza8)
    return loss.reshape(())


# speedup vs baseline: 95.0704x; 1.1117x over previous
import jax
import jax.numpy as jnp
from jax import lax
from jax.experimental import pallas as pl
from jax.experimental.pallas import tpu as pltpu
from jax.experimental.pallas import tpu_sc as plsc

NC, NS = 2, 16
NV = 100000
NP = 100352
E = 3200000
K = 16
W = 128
VB = 56
EPR = 25088
EP = EPR * W
_ZCH = NP // NS

_sc_mesh = dict(core_axis_name="c", subcore_axis_name="s",
                num_cores=NC, num_subcores=NS)
_sc_params = pltpu.CompilerParams(use_tc_tiling_on_sc=False)

_H_TROWS = EPR // (NC * NS)
_H_BATCH = _H_TROWS // VB


def _hist_body(nidx_hbm, out_hbm, idx_v, ones_v, zb_v, bins):
    c = lax.axis_index("c")
    s = lax.axis_index("s")

    def _fill_ones(i, _):
        ones_v[pl.ds(i * 16, 16)] = jnp.ones((16,), jnp.float32)
        return 0

    lax.fori_loop(0, W // 16, _fill_ones, 0)

    def _fill_zeros(i, _):
        zb_v[pl.ds(i * 16, 16)] = jnp.zeros((16,), jnp.float32)
        return 0

    lax.fori_loop(0, _ZCH // 16, _fill_zeros, 0)

    zbase = pl.multiple_of(s * _ZCH, 128)
    pltpu.sync_copy(zb_v, bins.at[pl.ds(zbase, _ZCH)])
    plsc.subcore_barrier()

    base = (c * NS + s) * _H_TROWS
    for b in range(_H_BATCH):
        r0 = pl.multiple_of(base + b * VB, 8)
        pltpu.sync_copy(nidx_hbm.at[pl.ds(r0, VB), :], idx_v)

        def _one_row(t, _):
            pltpu.sync_copy(ones_v, bins.at[idx_v.at[t]], add=True)
            return 0

        lax.fori_loop(0, VB, _one_row, 0)

    plsc.subcore_barrier()
    pltpu.sync_copy(bins.at[pl.ds(zbase, _ZCH)], out_hbm.at[c, 0, pl.ds(zbase, _ZCH)])


@jax.jit
def _hist_call(nidx2):
    return pl.kernel(
        _hist_body,
        out_type=jax.ShapeDtypeStruct((NC, 1, NP), jnp.float32),
        mesh=plsc.VectorSubcoreMesh(**_sc_mesh),
        compiler_params=_sc_params,
        scratch_types=[
            pltpu.VMEM((VB, W), jnp.int32),
            pltpu.VMEM((W,), jnp.float32),
            pltpu.VMEM((_ZCH,), jnp.float32),
            pltpu.VMEM_SHARED((NP,), jnp.float32),
        ],
    )(nidx2)


_GROUPS = NP // 128
_P_GB = 56
_P_GRID = _GROUPS // _P_GB


def _sel(q):
    m = lax.broadcasted_iota(jnp.int32, (128, 128), 0)
    l = lax.broadcasted_iota(jnp.int32, (128, 128), 1)
    return (m == 8 * q + l // 16).astype(jnp.float32)


def _prep_body(z_ref, dv_ref, y_ref, den_ref, sza_ref):
    i = pl.program_id(0)
    dv = dv_ref[0] + dv_ref[1]
    dvc = jnp.maximum(dv, 1.0)
    scale = lax.rsqrt(dvc)
    act = (dv > 0.0).astype(jnp.float32)
    den_p = jnp.zeros((1, 128), jnp.float32)
    sza_p = jnp.zeros((1, 128), jnp.float32)
    for q in range(16):
        sq = _sel(q)
        z = z_ref[:, q, :]
        z2 = z * z
        y_ref[:, q, :] = z * jnp.dot(scale, sq, preferred_element_type=jnp.float32)
        den_p += jnp.sum(z2 * jnp.dot(dvc, sq, preferred_element_type=jnp.float32),
                         axis=0, keepdims=True)
        sza_p += jnp.sum(z2 * jnp.dot(act, sq, preferred_element_type=jnp.float32),
                         axis=0, keepdims=True)

    @pl.when(i == 0)
    def _():
        den_ref[...] = den_p
        sza_ref[...] = sza_p

    @pl.when(i > 0)
    def _():
        den_ref[...] += den_p
        sza_ref[...] += sza_p


@jax.jit
def _prep_call(z4, dvp):
    return pl.pallas_call(
        _prep_body,
        grid=(_P_GRID,),
        in_specs=[
            pl.BlockSpec((_P_GB, 16, 128), lambda i: (i, 0, 0)),
            pl.BlockSpec((NC, _P_GB, 128), lambda i: (0, i, 0)),
        ],
        out_specs=[
            pl.BlockSpec((_P_GB, 16, 128), lambda i: (i, 0, 0)),
            pl.BlockSpec((1, 128), lambda i: (0, 0)),
            pl.BlockSpec((1, 128), lambda i: (0, 0)),
        ],
        out_shape=[
            jax.ShapeDtypeStruct((_GROUPS, 16, 128), jnp.float32),
            jax.ShapeDtypeStruct((1, 128), jnp.float32),
            jax.ShapeDtypeStruct((1, 128), jnp.float32),
        ],
    )(z4, dvp)


_S_TROWS = EPR // (NC * NS)
_S_BATCH = _S_TROWS // VB
_S_ZROWS = 784


_NBUF = 4


def _scatter_body(y_hbm, nidx_hbm, eidx_hbm, out_hbm, oute_hbm, nv, ev,
                  r0b, r1b, r2b, r3b, ones_v, zv, s0, s1, s2, s3, acc, ebins):
    c = lax.axis_index("c")
    s = lax.axis_index("s")
    g = c * NS + s
    rbufs = (r0b, r1b, r2b, r3b)
    sems = (s0, s1, s2, s3)

    def _fill_zeros(i, _):
        r0b[i, :] = jnp.zeros((16,), jnp.float32)
        return 0

    lax.fori_loop(0, W, _fill_zeros, 0)

    def _fill_ones(i, _):
        ones_v[pl.ds(i * 16, 16)] = jnp.ones((16,), jnp.float32)
        zv[pl.ds(i * 16, 16)] = jnp.zeros((16,), jnp.float32)
        return 0

    lax.fori_loop(0, W // 16, _fill_ones, 0)

    for t in range(_ZCH // W):
        z0 = pl.multiple_of(s * _ZCH + t * W, 8)
        pltpu.sync_copy(r0b, acc.at[pl.ds(z0, W), :])
        pltpu.sync_copy(zv, ebins.at[pl.ds(z0, W)])
    plsc.subcore_barrier()

    base = g * _S_TROWS

    def _batch(b, _):
        r0 = pl.multiple_of(base + b * VB, 8)
        pltpu.sync_copy(nidx_hbm.at[pl.ds(r0, VB), :], nv)
        pltpu.sync_copy(eidx_hbm.at[pl.ds(r0, VB), :], ev)
        for t in range(_NBUF):
            pltpu.async_copy(y_hbm.at[nv.at[t]], rbufs[t], sems[t])
        for t in range(VB):
            k = t % _NBUF
            pltpu.make_async_copy(y_hbm.at[nv.at[t]], rbufs[k], sems[k]).wait()
            pltpu.sync_copy(rbufs[k], acc.at[ev.at[t]], add=True)
            pltpu.sync_copy(ones_v, ebins.at[ev.at[t]], add=True)
            if t + _NBUF < VB:
                pltpu.async_copy(y_hbm.at[nv.at[t + _NBUF]], rbufs[k], sems[k])
        return 0

    lax.fori_loop(0, _S_BATCH, _batch, 0)

    plsc.subcore_barrier()
    for t in range(8):
        r0 = pl.multiple_of(s * _ZCH + t * _S_ZROWS, 8)
        pltpu.sync_copy(acc.at[pl.ds(r0, _S_ZROWS), :],
                        out_hbm.at[c, pl.ds(r0, _S_ZROWS), :])
    zbase = pl.multiple_of(s * _ZCH, 128)
    pltpu.sync_copy(ebins.at[pl.ds(zbase, _ZCH)], oute_hbm.at[c, 0, pl.ds(zbase, _ZCH)])


@jax.jit
def _scatter_call(y, nidx2, eidx2):
    return pl.kernel(
        _scatter_body,
        out_type=[
            jax.ShapeDtypeStruct((NC, NP, K), jnp.float32),
            jax.ShapeDtypeStruct((NC, 1, NP), jnp.float32),
        ],
        mesh=plsc.VectorSubcoreMesh(**_sc_mesh),
        compiler_params=_sc_params,
        scratch_types=[
            pltpu.VMEM((VB, W), jnp.int32),
            pltpu.VMEM((VB, W), jnp.int32),
            pltpu.VMEM((W, K), jnp.float32),
            pltpu.VMEM((W, K), jnp.float32),
            pltpu.VMEM((W, K), jnp.float32),
            pltpu.VMEM((W, K), jnp.float32),
            pltpu.VMEM((W,), jnp.float32),
            pltpu.VMEM((W,), jnp.float32),
            pltpu.SemaphoreType.DMA,
            pltpu.SemaphoreType.DMA,
            pltpu.SemaphoreType.DMA,
            pltpu.SemaphoreType.DMA,
            pltpu.VMEM_SHARED((NP, K), jnp.float32),
            pltpu.VMEM_SHARED((NP,), jnp.float32),
        ],
    )(y, nidx2, eidx2)


_F_GB = 56
_F_GRID = _GROUPS // _F_GB


def _final_body(p0_ref, p1_ref, de_ref, den_ref, sza_ref, out_ref, acc_ref):
    i = pl.program_id(0)

    @pl.when(i == 0)
    def _():
        acc_ref[...] = jnp.zeros_like(acc_ref)

    de = jnp.maximum(de_ref[0] + de_ref[1], 1.0)
    part = jnp.zeros((1, 128), jnp.float32)
    for q in range(16):
        de16 = jnp.dot(de, _sel(q), preferred_element_type=jnp.float32)
        ys = p0_ref[:, q, :] + p1_ref[:, q, :]
        part += jnp.sum(ys * ys / de16, axis=0, keepdims=True)
    acc_ref[...] += part

    @pl.when(i == pl.num_programs(0) - 1)
    def _():
        lf = lax.broadcasted_iota(jnp.int32, (128, 16), 0)
        cf = lax.broadcasted_iota(jnp.int32, (128, 16), 1)
        fold = (lf % 16 == cf).astype(jnp.float32)
        num16 = jnp.dot(sza_ref[...] - acc_ref[...], fold,
                        preferred_element_type=jnp.float32)
        den16 = jnp.dot(den_ref[...], fold, preferred_element_type=jnp.float32)
        rq = num16 / (den16 + 1e-8)
        out_ref[...] = (jnp.sum(rq) / 16.0).reshape(1, 1)


@jax.jit
def _final_call(p0, p1, dep, den8, sza8):
    return pl.pallas_call(
        _final_body,
        grid=(_F_GRID,),
        in_specs=[
            pl.BlockSpec((_F_GB, 16, 128), lambda i: (i, 0, 0)),
            pl.BlockSpec((_F_GB, 16, 128), lambda i: (i, 0, 0)),
            pl.BlockSpec((NC, _F_GB, 128), lambda i: (0, i, 0)),
            pl.BlockSpec((1, 128), lambda i: (0, 0)),
            pl.BlockSpec((1, 128), lambda i: (0, 0)),
        ],
        out_specs=pl.BlockSpec((1, 1), lambda i: (0, 0)),
        out_shape=jax.ShapeDtypeStruct((1, 1), jnp.float32),
        scratch_shapes=[pltpu.VMEM((1, 128), jnp.float32)],
    )(p0, p1, dep, den8, sza8)


def kernel(Z, hyperedge_index, num_nodes):
    del num_nodes
    pad = jnp.full((EP - E,), NV, jnp.int32)
    nidx2 = jnp.concatenate([hyperedge_index[0], pad]).reshape(EPR, W)
    eidx2 = jnp.concatenate([hyperedge_index[1], pad]).reshape(EPR, W)
    hist = _hist_call(nidx2)
    dvp = hist.reshape(NC, _GROUPS, 128)
    z4 = jnp.pad(Z, ((0, NP - NV), (0, 0))).reshape(_GROUPS, 16, 128)
    y4, den8, sza8 = _prep_call(z4, dvp)
    y = y4.reshape(NP, K)
    parts, ehist = _scatter_call(y, nidx2, eidx2)
    parts4 = parts.reshape(NC, _GROUPS, 16, 128)
    dep = ehist.reshape(NC, _GROUPS, 128)
    loss = _final_call(parts4[0], parts4[1], dep, den8, sza8)
    return loss.reshape(())


# speedup vs baseline: 102.7368x; 1.0806x over previous
import jax
import jax.numpy as jnp
from jax import lax
from jax.experimental import pallas as pl
from jax.experimental.pallas import tpu as pltpu
from jax.experimental.pallas import tpu_sc as plsc

NC, NS = 2, 16
NV = 100000
NP = 100352
E = 3200000
K = 16
W = 128
VB = 56
EPR = E // W
_ZCH = NP // NS

_T_FULL = 784
_T_SMALL = 728
_REM_BASE = 30 * _T_FULL + 2 * _T_SMALL
_REM = EPR - _REM_BASE


def _tile_base_nb(g):
    base = jnp.where(g < 30, g * _T_FULL, 30 * _T_FULL + (g - 30) * _T_SMALL)
    nb = jnp.where(g < 30, _T_FULL // VB, _T_SMALL // VB)
    return base, nb

_sc_mesh = dict(core_axis_name="c", subcore_axis_name="s",
                num_cores=NC, num_subcores=NS)
_sc_params = pltpu.CompilerParams(use_tc_tiling_on_sc=False)

def _hist_body(idx_hbm, out_hbm, idx_v, ones_v, zb_v, bins):
    c = lax.axis_index("c")
    s = lax.axis_index("s")
    g = c * NS + s

    def _fill_ones(i, _):
        ones_v[pl.ds(i * 16, 16)] = jnp.ones((16,), jnp.float32)
        return 0

    lax.fori_loop(0, W // 16, _fill_ones, 0)

    def _fill_zeros(i, _):
        zb_v[pl.ds(i * 16, 16)] = jnp.zeros((16,), jnp.float32)
        return 0

    lax.fori_loop(0, _ZCH // 16, _fill_zeros, 0)

    zbase = pl.multiple_of(s * _ZCH, 128)
    pltpu.sync_copy(zb_v, bins.at[pl.ds(zbase, _ZCH)])
    plsc.subcore_barrier()

    base, nb = _tile_base_nb(g)

    def _hist_rows(r0, n):
        pltpu.sync_copy(idx_hbm.at[0, pl.ds(r0, n), :], idx_v.at[pl.ds(0, n), :])

        def _one_row(t, _):
            pltpu.sync_copy(ones_v, bins.at[idx_v.at[t]], add=True)
            return 0

        lax.fori_loop(0, n, _one_row, 0)

    def _one_batch(b, _):
        _hist_rows(pl.multiple_of(base + b * VB, 8), VB)
        return 0

    lax.fori_loop(0, nb, _one_batch, 0)

    @pl.when(g == 30)
    def _():
        _hist_rows(_REM_BASE, _REM)

    plsc.subcore_barrier()
    pltpu.sync_copy(bins.at[pl.ds(zbase, _ZCH)], out_hbm.at[c, 0, pl.ds(zbase, _ZCH)])


@jax.jit
def _hist_call(idx3):
    return pl.kernel(
        _hist_body,
        out_type=jax.ShapeDtypeStruct((NC, 1, NP), jnp.float32),
        mesh=plsc.VectorSubcoreMesh(**_sc_mesh),
        compiler_params=_sc_params,
        scratch_types=[
            pltpu.VMEM((VB, W), jnp.int32),
            pltpu.VMEM((W,), jnp.float32),
            pltpu.VMEM((_ZCH,), jnp.float32),
            pltpu.VMEM_SHARED((NP,), jnp.float32),
        ],
    )(idx3)


_GROUPS = NP // 128
_P_GB = 56
_P_GRID = _GROUPS // _P_GB


def _sel(q):
    m = lax.broadcasted_iota(jnp.int32, (128, 128), 0)
    l = lax.broadcasted_iota(jnp.int32, (128, 128), 1)
    return (m == 8 * q + l // 16).astype(jnp.float32)


def _prep_body(z_ref, dv_ref, y_ref, den_ref, sza_ref):
    i = pl.program_id(0)
    dv = dv_ref[0] + dv_ref[1]
    dvc = jnp.maximum(dv, 1.0)
    scale = lax.rsqrt(dvc)
    act = (dv > 0.0).astype(jnp.float32)
    den_p = jnp.zeros((1, 128), jnp.float32)
    sza_p = jnp.zeros((1, 128), jnp.float32)
    for q in range(16):
        sq = _sel(q)
        z = z_ref[:, q, :]
        z2 = z * z
        y_ref[:, q, :] = z * jnp.dot(scale, sq, preferred_element_type=jnp.float32)
        den_p += jnp.sum(z2 * jnp.dot(dvc, sq, preferred_element_type=jnp.float32),
                         axis=0, keepdims=True)
        sza_p += jnp.sum(z2 * jnp.dot(act, sq, preferred_element_type=jnp.float32),
                         axis=0, keepdims=True)

    @pl.when(i == 0)
    def _():
        den_ref[...] = den_p
        sza_ref[...] = sza_p

    @pl.when(i > 0)
    def _():
        den_ref[...] += den_p
        sza_ref[...] += sza_p


@jax.jit
def _prep_call(z4, dvp):
    return pl.pallas_call(
        _prep_body,
        grid=(_P_GRID,),
        in_specs=[
            pl.BlockSpec((_P_GB, 16, 128), lambda i: (i, 0, 0)),
            pl.BlockSpec((NC, _P_GB, 128), lambda i: (0, i, 0)),
        ],
        out_specs=[
            pl.BlockSpec((_P_GB, 16, 128), lambda i: (i, 0, 0)),
            pl.BlockSpec((1, 128), lambda i: (0, 0)),
            pl.BlockSpec((1, 128), lambda i: (0, 0)),
        ],
        out_shape=[
            jax.ShapeDtypeStruct((_GROUPS, 16, 128), jnp.float32),
            jax.ShapeDtypeStruct((1, 128), jnp.float32),
            jax.ShapeDtypeStruct((1, 128), jnp.float32),
        ],
    )(z4, dvp)


_S_ZROWS = 784


_NBUF = 4


def _scatter_body(y_hbm, idx_hbm, out_hbm, oute_hbm, nv, ev,
                  r0b, r1b, r2b, r3b, ones_v, zv, s0, s1, s2, s3, acc, ebins):
    c = lax.axis_index("c")
    s = lax.axis_index("s")
    g = c * NS + s
    rbufs = (r0b, r1b, r2b, r3b)
    sems = (s0, s1, s2, s3)

    def _fill_zeros(i, _):
        r0b[i, :] = jnp.zeros((16,), jnp.float32)
        return 0

    lax.fori_loop(0, W, _fill_zeros, 0)

    def _fill_ones(i, _):
        ones_v[pl.ds(i * 16, 16)] = jnp.ones((16,), jnp.float32)
        zv[pl.ds(i * 16, 16)] = jnp.zeros((16,), jnp.float32)
        return 0

    lax.fori_loop(0, W // 16, _fill_ones, 0)

    for t in range(_ZCH // W):
        z0 = pl.multiple_of(s * _ZCH + t * W, 8)
        pltpu.sync_copy(r0b, acc.at[pl.ds(z0, W), :])
        pltpu.sync_copy(zv, ebins.at[pl.ds(z0, W)])
    plsc.subcore_barrier()

    base, nb = _tile_base_nb(g)

    def _pairs(r0, n):
        pltpu.sync_copy(idx_hbm.at[0, pl.ds(r0, n), :], nv.at[pl.ds(0, n), :])
        pltpu.sync_copy(idx_hbm.at[1, pl.ds(r0, n), :], ev.at[pl.ds(0, n), :])
        for t in range(_NBUF):
            pltpu.async_copy(y_hbm.at[nv.at[t]], rbufs[t], sems[t])
        for t in range(n):
            k = t % _NBUF
            pltpu.make_async_copy(y_hbm.at[nv.at[t]], rbufs[k], sems[k]).wait()
            pltpu.sync_copy(rbufs[k], acc.at[ev.at[t]], add=True)
            pltpu.sync_copy(ones_v, ebins.at[ev.at[t]], add=True)
            if t + _NBUF < n:
                pltpu.async_copy(y_hbm.at[nv.at[t + _NBUF]], rbufs[k], sems[k])

    def _batch(b, _):
        _pairs(pl.multiple_of(base + b * VB, 8), VB)
        return 0

    lax.fori_loop(0, nb, _batch, 0)

    @pl.when(g == 30)
    def _():
        _pairs(_REM_BASE, _REM)

    plsc.subcore_barrier()
    for t in range(8):
        r0 = pl.multiple_of(s * _ZCH + t * _S_ZROWS, 8)
        pltpu.sync_copy(acc.at[pl.ds(r0, _S_ZROWS), :],
                        out_hbm.at[c, pl.ds(r0, _S_ZROWS), :])
    zbase = pl.multiple_of(s * _ZCH, 128)
    pltpu.sync_copy(ebins.at[pl.ds(zbase, _ZCH)], oute_hbm.at[c, 0, pl.ds(zbase, _ZCH)])


@jax.jit
def _scatter_call(y, idx3):
    return pl.kernel(
        _scatter_body,
        out_type=[
            jax.ShapeDtypeStruct((NC, NP, K), jnp.float32),
            jax.ShapeDtypeStruct((NC, 1, NP), jnp.float32),
        ],
        mesh=plsc.VectorSubcoreMesh(**_sc_mesh),
        compiler_params=_sc_params,
        scratch_types=[
            pltpu.VMEM((VB, W), jnp.int32),
            pltpu.VMEM((VB, W), jnp.int32),
            pltpu.VMEM((W, K), jnp.float32),
            pltpu.VMEM((W, K), jnp.float32),
            pltpu.VMEM((W, K), jnp.float32),
            pltpu.VMEM((W, K), jnp.float32),
            pltpu.VMEM((W,), jnp.float32),
            pltpu.VMEM((W,), jnp.float32),
            pltpu.SemaphoreType.DMA,
            pltpu.SemaphoreType.DMA,
            pltpu.SemaphoreType.DMA,
            pltpu.SemaphoreType.DMA,
            pltpu.VMEM_SHARED((NP, K), jnp.float32),
            pltpu.VMEM_SHARED((NP,), jnp.float32),
        ],
    )(y, idx3)


_F_GB = 56
_F_GRID = _GROUPS // _F_GB


def _final_body(p0_ref, p1_ref, de_ref, den_ref, sza_ref, out_ref, acc_ref):
    i = pl.program_id(0)

    @pl.when(i == 0)
    def _():
        acc_ref[...] = jnp.zeros_like(acc_ref)

    de = jnp.maximum(de_ref[0] + de_ref[1], 1.0)
    part = jnp.zeros((1, 128), jnp.float32)
    for q in range(16):
        de16 = jnp.dot(de, _sel(q), preferred_element_type=jnp.float32)
        ys = p0_ref[:, q, :] + p1_ref[:, q, :]
        part += jnp.sum(ys * ys / de16, axis=0, keepdims=True)
    acc_ref[...] += part

    @pl.when(i == pl.num_programs(0) - 1)
    def _():
        lf = lax.broadcasted_iota(jnp.int32, (128, 16), 0)
        cf = lax.broadcasted_iota(jnp.int32, (128, 16), 1)
        fold = (lf % 16 == cf).astype(jnp.float32)
        num16 = jnp.dot(sza_ref[...] - acc_ref[...], fold,
                        preferred_element_type=jnp.float32)
        den16 = jnp.dot(den_ref[...], fold, preferred_element_type=jnp.float32)
        rq = num16 / (den16 + 1e-8)
        out_ref[...] = (jnp.sum(rq) / 16.0).reshape(1, 1)


@jax.jit
def _final_call(p0, p1, dep, den8, sza8):
    return pl.pallas_call(
        _final_body,
        grid=(_F_GRID,),
        in_specs=[
            pl.BlockSpec((_F_GB, 16, 128), lambda i: (i, 0, 0)),
            pl.BlockSpec((_F_GB, 16, 128), lambda i: (i, 0, 0)),
            pl.BlockSpec((NC, _F_GB, 128), lambda i: (0, i, 0)),
            pl.BlockSpec((1, 128), lambda i: (0, 0)),
            pl.BlockSpec((1, 128), lambda i: (0, 0)),
        ],
        out_specs=pl.BlockSpec((1, 1), lambda i: (0, 0)),
        out_shape=jax.ShapeDtypeStruct((1, 1), jnp.float32),
        scratch_shapes=[pltpu.VMEM((1, 128), jnp.float32)],
    )(p0, p1, dep, den8, sza8)


def kernel(Z, hyperedge_index, num_nodes):
    del num_nodes
    idx3 = hyperedge_index.reshape(2, EPR, W)
    hist = _hist_call(idx3)
    dvp = hist.reshape(NC, _GROUPS, 128)
    z4 = jnp.pad(Z, ((0, NP - NV), (0, 0))).reshape(_GROUPS, 16, 128)
    y4, den8, sza8 = _prep_call(z4, dvp)
    y = y4.reshape(NP, K)
    parts, ehist = _scatter_call(y, idx3)
    parts4 = parts.reshape(NC, _GROUPS, 16, 128)
    dep = ehist.reshape(NC, _GROUPS, 128)
    loss = _final_call(parts4[0], parts4[1], dep, den8, sza8)
    return loss.reshape(())


# speedup vs baseline: 103.4817x; 1.0073x over previous
import jax
import jax.numpy as jnp
from jax import lax
from jax.experimental import pallas as pl
from jax.experimental.pallas import tpu as pltpu
from jax.experimental.pallas import tpu_sc as plsc

NC, NS = 2, 16
NV = 100000
NP = 100352
E = 3200000
K = 16
W = 128
VB = 56
EPR = E // W
_ZCH = NP // NS

_T_FULL = 784
_T_SMALL = 728
_REM_BASE = 30 * _T_FULL + 2 * _T_SMALL
_REM = EPR - _REM_BASE


def _tile_base_nb(g):
    base = jnp.where(g < 30, g * _T_FULL, 30 * _T_FULL + (g - 30) * _T_SMALL)
    nb = jnp.where(g < 30, _T_FULL // VB, _T_SMALL // VB)
    return base, nb

_sc_mesh = dict(core_axis_name="c", subcore_axis_name="s",
                num_cores=NC, num_subcores=NS)
_sc_params = pltpu.CompilerParams(use_tc_tiling_on_sc=False)

def _hist_body(idx_hbm, out_hbm, idx_v, ones_v, zb_v, bins):
    c = lax.axis_index("c")
    s = lax.axis_index("s")
    g = c * NS + s

    def _fill_ones(i, _):
        ones_v[pl.ds(i * 16, 16)] = jnp.ones((16,), jnp.float32)
        return 0

    lax.fori_loop(0, W // 16, _fill_ones, 0)

    def _fill_zeros(i, _):
        zb_v[pl.ds(i * 16, 16)] = jnp.zeros((16,), jnp.float32)
        return 0

    lax.fori_loop(0, _ZCH // 16, _fill_zeros, 0)

    zbase = pl.multiple_of(s * _ZCH, 128)
    pltpu.sync_copy(zb_v, bins.at[pl.ds(zbase, _ZCH)])
    plsc.subcore_barrier()

    base, nb = _tile_base_nb(g)

    def _hist_rows(r0, n):
        pltpu.sync_copy(idx_hbm.at[0, pl.ds(r0, n), :], idx_v.at[pl.ds(0, n), :])

        def _one_row(t, _):
            pltpu.sync_copy(ones_v, bins.at[idx_v.at[t]], add=True)
            return 0

        lax.fori_loop(0, n, _one_row, 0)

    def _one_batch(b, _):
        _hist_rows(pl.multiple_of(base + b * VB, 8), VB)
        return 0

    lax.fori_loop(0, nb, _one_batch, 0)

    @pl.when(g == 30)
    def _():
        _hist_rows(_REM_BASE, _REM)

    plsc.subcore_barrier()
    pltpu.sync_copy(bins.at[pl.ds(zbase, _ZCH)], out_hbm.at[c, 0, pl.ds(zbase, _ZCH)])


@jax.jit
def _hist_call(idx3):
    return pl.kernel(
        _hist_body,
        out_type=jax.ShapeDtypeStruct((NC, 1, NP), jnp.float32),
        mesh=plsc.VectorSubcoreMesh(**_sc_mesh),
        compiler_params=_sc_params,
        scratch_types=[
            pltpu.VMEM((VB, W), jnp.int32),
            pltpu.VMEM((W,), jnp.float32),
            pltpu.VMEM((_ZCH,), jnp.float32),
            pltpu.VMEM_SHARED((NP,), jnp.float32),
        ],
    )(idx3)


_GROUPS = NP // 128
_P_GB = 56
_P_GRID = _GROUPS // _P_GB


def _sel(q):
    m = lax.broadcasted_iota(jnp.int32, (128, 128), 0)
    l = lax.broadcasted_iota(jnp.int32, (128, 128), 1)
    return (m == 8 * q + l // 16).astype(jnp.float32)


def _prep_body(z_ref, dv_ref, y_ref, den_ref, sza_ref):
    i = pl.program_id(0)
    dv = dv_ref[0] + dv_ref[1]
    dvc = jnp.maximum(dv, 1.0)
    scale = lax.rsqrt(dvc)
    act = (dv > 0.0).astype(jnp.float32)
    den_p = jnp.zeros((1, 128), jnp.float32)
    sza_p = jnp.zeros((1, 128), jnp.float32)
    for q in range(16):
        sq = _sel(q)
        z = z_ref[:, q, :]
        z2 = z * z
        y_ref[:, q, :] = z * jnp.dot(scale, sq, preferred_element_type=jnp.float32)
        den_p += jnp.sum(z2 * jnp.dot(dvc, sq, preferred_element_type=jnp.float32),
                         axis=0, keepdims=True)
        sza_p += jnp.sum(z2 * jnp.dot(act, sq, preferred_element_type=jnp.float32),
                         axis=0, keepdims=True)

    @pl.when(i == 0)
    def _():
        den_ref[...] = den_p
        sza_ref[...] = sza_p

    @pl.when(i > 0)
    def _():
        den_ref[...] += den_p
        sza_ref[...] += sza_p


@jax.jit
def _prep_call(z4, dvp):
    return pl.pallas_call(
        _prep_body,
        grid=(_P_GRID,),
        in_specs=[
            pl.BlockSpec((_P_GB, 16, 128), lambda i: (i, 0, 0)),
            pl.BlockSpec((NC, _P_GB, 128), lambda i: (0, i, 0)),
        ],
        out_specs=[
            pl.BlockSpec((_P_GB, 16, 128), lambda i: (i, 0, 0)),
            pl.BlockSpec((1, 128), lambda i: (0, 0)),
            pl.BlockSpec((1, 128), lambda i: (0, 0)),
        ],
        out_shape=[
            jax.ShapeDtypeStruct((_GROUPS, 16, 128), jnp.float32),
            jax.ShapeDtypeStruct((1, 128), jnp.float32),
            jax.ShapeDtypeStruct((1, 128), jnp.float32),
        ],
    )(z4, dvp)


_S_ZROWS = 784


_NBUF = 5
_EVH = VB // 2


def _scatter_body(y_hbm, idx_hbm, out_hbm, oute_hbm, nv, ev,
                  r0b, r1b, r2b, r3b, r4b, ones_v, zv,
                  s0, s1, s2, s3, s4, acc, ebins):
    c = lax.axis_index("c")
    s = lax.axis_index("s")
    g = c * NS + s
    rbufs = (r0b, r1b, r2b, r3b, r4b)
    sems = (s0, s1, s2, s3, s4)

    def _fill_zeros(i, _):
        r0b[i, :] = jnp.zeros((16,), jnp.float32)
        return 0

    lax.fori_loop(0, W, _fill_zeros, 0)

    def _fill_ones(i, _):
        ones_v[pl.ds(i * 16, 16)] = jnp.ones((16,), jnp.float32)
        zv[pl.ds(i * 16, 16)] = jnp.zeros((16,), jnp.float32)
        return 0

    lax.fori_loop(0, W // 16, _fill_ones, 0)

    for t in range(_ZCH // W):
        z0 = pl.multiple_of(s * _ZCH + t * W, 8)
        pltpu.sync_copy(r0b, acc.at[pl.ds(z0, W), :])
        pltpu.sync_copy(zv, ebins.at[pl.ds(z0, W)])
    plsc.subcore_barrier()

    base, nb = _tile_base_nb(g)

    def _pairs(r0, n):
        h = min(n, _EVH)
        pltpu.sync_copy(idx_hbm.at[0, pl.ds(r0, n), :], nv.at[pl.ds(0, n), :])
        pltpu.sync_copy(idx_hbm.at[1, pl.ds(r0, h), :], ev.at[pl.ds(0, h), :])
        for t in range(_NBUF):
            pltpu.async_copy(y_hbm.at[nv.at[t]], rbufs[t], sems[t])
        for t in range(n):
            if t == h and n > h:
                pltpu.sync_copy(idx_hbm.at[1, pl.ds(r0 + h, n - h), :],
                                ev.at[pl.ds(0, n - h), :])
            k = t % _NBUF
            pltpu.make_async_copy(y_hbm.at[nv.at[t]], rbufs[k], sems[k]).wait()
            e_t = ev.at[t] if t < h else ev.at[t - h]
            pltpu.sync_copy(rbufs[k], acc.at[e_t], add=True)
            pltpu.sync_copy(ones_v, ebins.at[e_t], add=True)
            if t + _NBUF < n:
                pltpu.async_copy(y_hbm.at[nv.at[t + _NBUF]], rbufs[k], sems[k])

    def _batch(b, _):
        _pairs(pl.multiple_of(base + b * VB, 8), VB)
        return 0

    lax.fori_loop(0, nb, _batch, 0)

    @pl.when(g == 30)
    def _():
        _pairs(_REM_BASE, _REM)

    plsc.subcore_barrier()
    for t in range(8):
        r0 = pl.multiple_of(s * _ZCH + t * _S_ZROWS, 8)
        pltpu.sync_copy(acc.at[pl.ds(r0, _S_ZROWS), :],
                        out_hbm.at[c, pl.ds(r0, _S_ZROWS), :])
    zbase = pl.multiple_of(s * _ZCH, 128)
    pltpu.sync_copy(ebins.at[pl.ds(zbase, _ZCH)], oute_hbm.at[c, 0, pl.ds(zbase, _ZCH)])


@jax.jit
def _scatter_call(y, idx3):
    return pl.kernel(
        _scatter_body,
        out_type=[
            jax.ShapeDtypeStruct((NC, NP, K), jnp.float32),
            jax.ShapeDtypeStruct((NC, 1, NP), jnp.float32),
        ],
        mesh=plsc.VectorSubcoreMesh(**_sc_mesh),
        compiler_params=_sc_params,
        scratch_types=[
            pltpu.VMEM((VB, W), jnp.int32),
            pltpu.VMEM((_EVH, W), jnp.int32),
            pltpu.VMEM((W, K), jnp.float32),
            pltpu.VMEM((W, K), jnp.float32),
            pltpu.VMEM((W, K), jnp.float32),
            pltpu.VMEM((W, K), jnp.float32),
            pltpu.VMEM((W, K), jnp.float32),
            pltpu.VMEM((W,), jnp.float32),
            pltpu.VMEM((W,), jnp.float32),
            pltpu.SemaphoreType.DMA,
            pltpu.SemaphoreType.DMA,
            pltpu.SemaphoreType.DMA,
            pltpu.SemaphoreType.DMA,
            pltpu.SemaphoreType.DMA,
            pltpu.VMEM_SHARED((NP, K), jnp.float32),
            pltpu.VMEM_SHARED((NP,), jnp.float32),
        ],
    )(y, idx3)


_F_GB = 56
_F_GRID = _GROUPS // _F_GB


def _final_body(p0_ref, p1_ref, de_ref, den_ref, sza_ref, out_ref, acc_ref):
    i = pl.program_id(0)

    @pl.when(i == 0)
    def _():
        acc_ref[...] = jnp.zeros_like(acc_ref)

    de = jnp.maximum(de_ref[0] + de_ref[1], 1.0)
    part = jnp.zeros((1, 128), jnp.float32)
    for q in range(16):
        de16 = jnp.dot(de, _sel(q), preferred_element_type=jnp.float32)
        ys = p0_ref[:, q, :] + p1_ref[:, q, :]
        part += jnp.sum(ys * ys / de16, axis=0, keepdims=True)
    acc_ref[...] += part

    @pl.when(i == pl.num_programs(0) - 1)
    def _():
        lf = lax.broadcasted_iota(jnp.int32, (128, 16), 0)
        cf = lax.broadcasted_iota(jnp.int32, (128, 16), 1)
        fold = (lf % 16 == cf).astype(jnp.float32)
        num16 = jnp.dot(sza_ref[...] - acc_ref[...], fold,
                        preferred_element_type=jnp.float32)
        den16 = jnp.dot(den_ref[...], fold, preferred_element_type=jnp.float32)
        rq = num16 / (den16 + 1e-8)
        out_ref[...] = (jnp.sum(rq) / 16.0).reshape(1, 1)


@jax.jit
def _final_call(p0, p1, dep, den8, sza8):
    return pl.pallas_call(
        _final_body,
        grid=(_F_GRID,),
        in_specs=[
            pl.BlockSpec((_F_GB, 16, 128), lambda i: (i, 0, 0)),
            pl.BlockSpec((_F_GB, 16, 128), lambda i: (i, 0, 0)),
            pl.BlockSpec((NC, _F_GB, 128), lambda i: (0, i, 0)),
            pl.BlockSpec((1, 128), lambda i: (0, 0)),
            pl.BlockSpec((1, 128), lambda i: (0, 0)),
        ],
        out_specs=pl.BlockSpec((1, 1), lambda i: (0, 0)),
        out_shape=jax.ShapeDtypeStruct((1, 1), jnp.float32),
        scratch_shapes=[pltpu.VMEM((1, 128), jnp.float32)],
    )(p0, p1, dep, den8, sza8)


def kernel(Z, hyperedge_index, num_nodes):
    del num_nodes
    idx3 = hyperedge_index.reshape(2, EPR, W)
    hist = _hist_call(idx3)
    dvp = hist.reshape(NC, _GROUPS, 128)
    z4 = jnp.pad(Z, ((0, NP - NV), (0, 0))).reshape(_GROUPS, 16, 128)
    y4, den8, sza8 = _prep_call(z4, dvp)
    y = y4.reshape(NP, K)
    parts, ehist = _scatter_call(y, idx3)
    parts4 = parts.reshape(NC, _GROUPS, 16, 128)
    dep = ehist.reshape(NC, _GROUPS, 128)
    loss = _final_call(parts4[0], parts4[1], dep, den8, sza8)
    return loss.reshape(())
